# baseline (speedup 1.0000x reference)
import functools

import jax
import jax.numpy as jnp
from jax import lax
from jax.experimental import pallas as pl
from jax.experimental.pallas import tpu as pltpu
from jax.experimental.pallas import tpu_sc as plsc

E = 8
TOPK = 2
D = 768
F = 768
T = 2048
A = T * TOPK

B = 256
NB = A // B + E
RMAX = NB * B

NW = 32
RPW = RMAX // NW
TPW = T // NW
DW = D // 2

_NEG = -1e30


def _router_body(x_ref, wg_ref, pos_ref, wts_ref, end_ref):
    x = x_ref[...]
    logits = jnp.dot(x, wg_ref[...], preferred_element_type=jnp.float32)
    idx = lax.broadcasted_iota(jnp.int32, (T, E), 1)
    m1 = jnp.max(logits, axis=1, keepdims=True)
    i1 = jnp.min(jnp.where(logits == m1, idx, E), axis=1, keepdims=True)
    l2 = jnp.where(idx == i1, _NEG, logits)
    m2 = jnp.max(l2, axis=1, keepdims=True)
    i2 = jnp.min(jnp.where(l2 == m2, idx, E), axis=1, keepdims=True)
    wa = jax.nn.sigmoid(m1 - m2)

    oh1 = (idx == i1).astype(jnp.float32)
    oh2 = (idx == i2).astype(jnp.float32)
    onehot = oh1 + oh2

    CB = 512
    r = lax.broadcasted_iota(jnp.int32, (CB, CB), 0)
    c = lax.broadcasted_iota(jnp.int32, (CB, CB), 1)
    tri = (c < r).astype(jnp.float32)
    run = jnp.zeros((1, E), jnp.float32)
    parts = []
    for bi in range(T // CB):
        ab = onehot[bi * CB:(bi + 1) * CB, :]
        parts.append(jnp.dot(tri, ab, preferred_element_type=jnp.float32) + run)
        run = run + jnp.sum(ab, axis=0, keepdims=True)
    cnt = jnp.concatenate(parts, axis=0)

    seg = jnp.floor((run + (B - 1)) * (1.0 / B)) * B
    er = lax.broadcasted_iota(jnp.int32, (E, E), 0)
    ec = lax.broadcasted_iota(jnp.int32, (E, E), 1)
    tri8 = (er < ec).astype(jnp.float32)
    off = jnp.dot(seg, tri8, preferred_element_type=jnp.float32)

    posc = cnt + off
    p1 = jnp.sum(oh1 * posc, axis=1, keepdims=True)
    p2 = jnp.sum(oh2 * posc, axis=1, keepdims=True)
    pos_ref[:, 0:1] = p1.astype(jnp.int32)
    pos_ref[:, 1:2] = p2.astype(jnp.int32)
    wts_ref[:, 0:1] = wa
    wts_ref[:, 1:2] = 1.0 - wa
    end_ref[...] = (off + seg).astype(jnp.int32)


def _router(x, wg):
    return pl.pallas_call(
        _router_body,
        in_specs=[
            pl.BlockSpec((T, D), lambda: (0, 0)),
            pl.BlockSpec((D, E), lambda: (0, 0)),
        ],
        out_specs=[
            pl.BlockSpec((T, TOPK), lambda: (0, 0)),
            pl.BlockSpec((T, TOPK), lambda: (0, 0)),
            pl.BlockSpec((1, E), lambda: (0, 0)),
        ],
        out_shape=[
            jax.ShapeDtypeStruct((T, TOPK), jnp.int32),
            jax.ShapeDtypeStruct((T, TOPK), jnp.float32),
            jax.ShapeDtypeStruct((1, E), jnp.int32),
        ],
    )(x, wg)


APS = A // 16
ZPS = RMAX // 16


@functools.cache
def _sc_dispatch():
    mesh = plsc.VectorSubcoreMesh(core_axis_name="c", subcore_axis_name="s")
    return functools.partial(
        pl.kernel,
        mesh=mesh,
        compiler_params=pltpu.CompilerParams(needs_layout_passes=False),
        out_type=[
            jax.ShapeDtypeStruct((RMAX, DW), jnp.int32),
            jax.ShapeDtypeStruct((RMAX,), jnp.float32),
        ],
        scratch_types=[
            pltpu.VMEM((APS,), jnp.int32),
            pltpu.VMEM((APS,), jnp.float32),
            pltpu.VMEM((APS,), jnp.int32),
            pltpu.VMEM((ZPS,), jnp.int32),
            pltpu.VMEM((ZPS,), jnp.float32),
            pltpu.VMEM((RPW,), jnp.int32),
            pltpu.VMEM((RPW, DW), jnp.int32),
            pltpu.VMEM_SHARED((RMAX,), jnp.int32),
            pltpu.VMEM_SHARED((RMAX,), jnp.float32),
            pltpu.SemaphoreType.DMA,
        ],
    )(_dispatch_body)


def _dispatch_body(x_hbm, pos_hbm, wts_hbm, xs_hbm, scale_hbm,
                   pos_v, wts_v, tok_v, zi_v, zf_v, r2t_v,
                   rows_v, r2t_sh, scale_sh, sem):
    cid = lax.axis_index("c")
    sid = lax.axis_index("s")
    wid = sid * 2 + cid

    abase = sid * APS
    pltpu.sync_copy(pos_hbm.at[pl.ds(abase, APS)], pos_v)
    pltpu.sync_copy(wts_hbm.at[pl.ds(abase, APS)], wts_v)

    zi = jnp.zeros((16,), jnp.int32)
    zf = jnp.zeros((16,), jnp.float32)
    lane = lax.iota(jnp.int32, 16)
    for j in range(ZPS // 16):
        zi_v[pl.ds(j * 16, 16)] = zi
        zf_v[pl.ds(j * 16, 16)] = zf
    for j in range(APS // 16):
        tok_v[pl.ds(j * 16, 16)] = lax.shift_right_logical(
            abase + j * 16 + lane, 1)

    pltpu.sync_copy(zi_v, r2t_sh.at[pl.ds(sid * ZPS, ZPS)])
    pltpu.sync_copy(zf_v, scale_sh.at[pl.ds(sid * ZPS, ZPS)])
    plsc.subcore_barrier()

    pltpu.sync_copy(tok_v, r2t_sh.at[pos_v])
    pltpu.sync_copy(wts_v, scale_sh.at[pos_v])
    plsc.subcore_barrier()

    @pl.when(wid == 0)
    def _():
        pltpu.sync_copy(scale_sh, scale_hbm)

    base = wid * RPW
    pltpu.sync_copy(r2t_sh.at[pl.ds(base, RPW)], r2t_v)
    H = RPW // 2
    h0 = pltpu.async_copy(x_hbm.at[r2t_v.at[pl.ds(0, H)]],
                          rows_v.at[pl.ds(0, H)], sem)
    h1 = pltpu.async_copy(x_hbm.at[r2t_v.at[pl.ds(H, H)]],
                          rows_v.at[pl.ds(H, H)], sem)
    h0.wait()
    h1.wait()
    pltpu.sync_copy(rows_v, xs_hbm.at[pl.ds(base, RPW)])


def _gemm_body(end_ref, xs_ref, scale_ref, w1_ref, w2_ref, ys_ref):
    b = pl.program_id(0)
    nbu = end_ref[E - 1] // B

    @pl.when(b < nbu)
    def _():
        xb = xs_ref[...]
        w1e = w1_ref[0].astype(jnp.bfloat16)
        h = lax.dot_general(xb, w1e, (((1,), (1,)), ((), ())),
                            preferred_element_type=jnp.float32)
        g = h[:, :F]
        u = h[:, F:]
        act = (g * jax.nn.sigmoid(g) * u).astype(jnp.bfloat16)
        y = jnp.dot(act, w2_ref[0].astype(jnp.bfloat16),
                    preferred_element_type=jnp.float32)
        ys_ref[...] = (y * scale_ref[...]).astype(jnp.bfloat16)


def _row_block(b, end_ref):
    nbu = end_ref[E - 1] // B
    return jnp.minimum(b, nbu - 1)


def _grp(b, end_ref):
    g = jnp.int32(0)
    for e in range(E):
        g = g + (end_ref[e] <= b * B).astype(jnp.int32)
    return jnp.minimum(g, E - 1)


def _gemm(endv, xs, scale, w1, w2):
    grid_spec = pltpu.PrefetchScalarGridSpec(
        num_scalar_prefetch=1,
        grid=(NB,),
        in_specs=[
            pl.BlockSpec((B, D), lambda b, end_ref: (_row_block(b, end_ref), 0)),
            pl.BlockSpec((B, 1), lambda b, end_ref: (_row_block(b, end_ref), 0)),
            pl.BlockSpec((1, 2 * F, D), lambda b, end_ref: (_grp(b, end_ref), 0, 0)),
            pl.BlockSpec((1, F, D), lambda b, end_ref: (_grp(b, end_ref), 0, 0)),
        ],
        out_specs=pl.BlockSpec((B, D), lambda b, end_ref: (_row_block(b, end_ref), 0)),
    )
    return pl.pallas_call(
        _gemm_body,
        grid_spec=grid_spec,
        out_shape=jax.ShapeDtypeStruct((RMAX, D), jnp.bfloat16),
    )(endv, xs, scale, w1, w2)


@functools.cache
def _sc_gather_pairs():
    mesh = plsc.VectorSubcoreMesh(core_axis_name="c", subcore_axis_name="s")
    return functools.partial(
        pl.kernel,
        mesh=mesh,
        out_type=jax.ShapeDtypeStruct((A, DW), jnp.int32),
        scratch_types=[
            pltpu.VMEM((TOPK * TPW,), jnp.int32),
            pltpu.VMEM((TOPK * TPW, DW), jnp.int32),
            pltpu.SemaphoreType.DMA,
        ],
    )(_gather_pairs_body)


def _gather_pairs_body(ys_hbm, pos_hbm, g_hbm, pos_v, buf_v, sem):
    wid = lax.axis_index("s") * 2 + lax.axis_index("c")
    abase = wid * TOPK * TPW
    pltpu.sync_copy(pos_hbm.at[pl.ds(abase, TOPK * TPW)], pos_v)
    pltpu.async_copy(ys_hbm.at[pos_v], buf_v, sem).wait()
    pltpu.sync_copy(buf_v, g_hbm.at[pl.ds(abase, TOPK * TPW)])


def _pairadd_body(g_ref, out_ref):
    a = g_ref[:, 0, :].astype(jnp.float32)
    b = g_ref[:, 1, :].astype(jnp.float32)
    out_ref[...] = a + b


def _pairadd(g16):
    return pl.pallas_call(
        _pairadd_body,
        in_specs=[pl.BlockSpec((T, TOPK, D), lambda: (0, 0, 0))],
        out_specs=pl.BlockSpec((T, D), lambda: (0, 0)),
        out_shape=jax.ShapeDtypeStruct((T, D), jnp.float32),
    )(g16)


@jax.jit
def kernel(hidden_states, Wg, w1, w2):
    pos, wts, endr = _router(hidden_states, Wg)
    posf = pos.reshape(A)
    wtsf = wts.reshape(A)
    endv = endr.reshape(E)
    x16 = hidden_states.astype(jnp.bfloat16)
    xi = lax.bitcast_convert_type(x16.reshape(T, DW, 2), jnp.int32)
    xsi, scale = _sc_dispatch()(xi, posf, wtsf)
    xs16 = lax.bitcast_convert_type(xsi, jnp.bfloat16).reshape(RMAX, D)
    ys = _gemm(endv, xs16, scale.reshape(RMAX, 1), w1, w2)
    yi = lax.bitcast_convert_type(ys.reshape(RMAX, DW, 2), jnp.int32)
    gi = _sc_gather_pairs()(yi, posf)
    g16 = lax.bitcast_convert_type(gi, jnp.bfloat16).reshape(T, TOPK, D)
    return _pairadd(g16)


# speedup vs baseline: 10.4686x; 10.4686x over previous
import jax
import jax.numpy as jnp
from jax import lax
from jax.experimental import pallas as pl
from jax.experimental.pallas import tpu as pltpu

E = 8
TOPK = 2
D = 768
F = 768
T = 2048
A = T * TOPK

B = 256
NB = A // B + E
RMAX = NB * B

TB = 512
RC = 512

_NEG = -1e30


def _router_body(x_ref, wg_ref, pos_ref, wts_ref, end_ref, x16_ref):
    x = x_ref[...]
    x16_ref[...] = x.astype(jnp.bfloat16)
    logits = jnp.dot(x, wg_ref[...], preferred_element_type=jnp.float32)
    idx = lax.broadcasted_iota(jnp.int32, (T, E), 1)
    m1 = jnp.max(logits, axis=1, keepdims=True)
    i1 = jnp.min(jnp.where(logits == m1, idx, E), axis=1, keepdims=True)
    l2 = jnp.where(idx == i1, _NEG, logits)
    m2 = jnp.max(l2, axis=1, keepdims=True)
    i2 = jnp.min(jnp.where(l2 == m2, idx, E), axis=1, keepdims=True)
    wa = jax.nn.sigmoid(m1 - m2)

    oh1 = (idx == i1).astype(jnp.float32)
    oh2 = (idx == i2).astype(jnp.float32)
    onehot = oh1 + oh2

    CB = 512
    r = lax.broadcasted_iota(jnp.int32, (CB, CB), 0)
    c = lax.broadcasted_iota(jnp.int32, (CB, CB), 1)
    tri = (c < r).astype(jnp.float32)
    run = jnp.zeros((1, E), jnp.float32)
    parts = []
    for bi in range(T // CB):
        ab = onehot[bi * CB:(bi + 1) * CB, :]
        parts.append(jnp.dot(tri, ab, preferred_element_type=jnp.float32) + run)
        run = run + jnp.sum(ab, axis=0, keepdims=True)
    cnt = jnp.concatenate(parts, axis=0)

    seg = jnp.floor((run + (B - 1)) * (1.0 / B)) * B
    er = lax.broadcasted_iota(jnp.int32, (E, E), 0)
    ec = lax.broadcasted_iota(jnp.int32, (E, E), 1)
    tri8 = (er < ec).astype(jnp.float32)
    off = jnp.dot(seg, tri8, preferred_element_type=jnp.float32)

    posc = cnt + off
    p1 = jnp.sum(oh1 * posc, axis=1, keepdims=True)
    p2 = jnp.sum(oh2 * posc, axis=1, keepdims=True)
    pos_ref[:, 0:1] = p1.astype(jnp.int32)
    pos_ref[:, 1:2] = p2.astype(jnp.int32)
    wts_ref[:, 0:1] = wa
    wts_ref[:, 1:2] = 1.0 - wa
    end_ref[...] = (off + seg).astype(jnp.int32)


def _router(x, wg):
    return pl.pallas_call(
        _router_body,
        in_specs=[
            pl.BlockSpec((T, D), lambda: (0, 0)),
            pl.BlockSpec((D, E), lambda: (0, 0)),
        ],
        out_specs=[
            pl.BlockSpec((T, TOPK), lambda: (0, 0)),
            pl.BlockSpec((T, TOPK), lambda: (0, 0)),
            pl.BlockSpec((1, E), lambda: (0, 0)),
            pl.BlockSpec((T, D), lambda: (0, 0)),
        ],
        out_shape=[
            jax.ShapeDtypeStruct((T, TOPK), jnp.int32),
            jax.ShapeDtypeStruct((T, TOPK), jnp.float32),
            jax.ShapeDtypeStruct((1, E), jnp.int32),
            jax.ShapeDtypeStruct((T, D), jnp.bfloat16),
        ],
    )(x, wg)


def _gemm_body(end_ref, pos_ref, x16_ref, w1_ref, w2_ref, ys_ref):
    b = pl.program_id(0)
    nbu = end_ref[0, E - 1] // B

    @pl.when(b < nbu)
    def _():
        ri = lax.broadcasted_iota(jnp.int32, (T, B), 1) + b * B
        p0 = pos_ref[:, 0:1]
        p1 = pos_ref[:, 1:2]
        pt = ((p0 == ri) | (p1 == ri)).astype(jnp.bfloat16)
        xs = lax.dot_general(pt, x16_ref[...], (((0,), (0,)), ((), ())),
                             preferred_element_type=jnp.float32)
        xb = xs.astype(jnp.bfloat16)
        w1e = w1_ref[0].astype(jnp.bfloat16)
        h = lax.dot_general(xb, w1e, (((1,), (1,)), ((), ())),
                            preferred_element_type=jnp.float32)
        g = h[:, :F]
        u = h[:, F:]
        act = (g * jax.nn.sigmoid(g) * u).astype(jnp.bfloat16)
        y = jnp.dot(act, w2_ref[0].astype(jnp.bfloat16),
                    preferred_element_type=jnp.float32)
        ys_ref[...] = y.astype(jnp.bfloat16)

    @pl.when(b >= nbu)
    def _zero():
        ys_ref[...] = jnp.zeros((B, D), jnp.bfloat16)


def _grp(b, end_ref):
    g = jnp.int32(0)
    for e in range(E):
        g = g + (end_ref[0, e] <= b * B).astype(jnp.int32)
    return jnp.minimum(g, E - 1)


def _gemm(endr, pos, x16, w1, w2):
    grid_spec = pltpu.PrefetchScalarGridSpec(
        num_scalar_prefetch=1,
        grid=(NB,),
        in_specs=[
            pl.BlockSpec((T, TOPK), lambda b, end_ref: (0, 0)),
            pl.BlockSpec((T, D), lambda b, end_ref: (0, 0)),
            pl.BlockSpec((1, 2 * F, D), lambda b, end_ref: (_grp(b, end_ref), 0, 0)),
            pl.BlockSpec((1, F, D), lambda b, end_ref: (_grp(b, end_ref), 0, 0)),
        ],
        out_specs=pl.BlockSpec((B, D), lambda b, end_ref: (b, 0)),
    )
    return pl.pallas_call(
        _gemm_body,
        grid_spec=grid_spec,
        out_shape=jax.ShapeDtypeStruct((RMAX, D), jnp.bfloat16),
    )(endr, pos, x16, w1, w2)


def _combine_body(pos_ref, wts_ref, ys_ref, out_ref):
    p0 = pos_ref[:, 0:1]
    p1 = pos_ref[:, 1:2]
    wa = wts_ref[:, 0:1]
    wb = wts_ref[:, 1:2]
    acc = jnp.zeros((TB, D), jnp.float32)
    for rc in range(RMAX // RC):
        ri = lax.broadcasted_iota(jnp.int32, (TB, RC), 1) + rc * RC
        cm = (jnp.where(p0 == ri, wa, 0.0)
              + jnp.where(p1 == ri, wb, 0.0)).astype(jnp.bfloat16)
        acc = acc + jnp.dot(cm, ys_ref[rc * RC:(rc + 1) * RC, :],
                            preferred_element_type=jnp.float32)
    out_ref[...] = acc


def _combine(pos, wts, ys):
    return pl.pallas_call(
        _combine_body,
        grid=(T // TB,),
        in_specs=[
            pl.BlockSpec((TB, TOPK), lambda tb: (tb, 0)),
            pl.BlockSpec((TB, TOPK), lambda tb: (tb, 0)),
            pl.BlockSpec((RMAX, D), lambda tb: (0, 0)),
        ],
        out_specs=pl.BlockSpec((TB, D), lambda tb: (tb, 0)),
        out_shape=jax.ShapeDtypeStruct((T, D), jnp.float32),
    )(pos, wts, ys)


@jax.jit
def kernel(hidden_states, Wg, w1, w2):
    pos, wts, endr, x16 = _router(hidden_states, Wg)
    ys = _gemm(endr, pos, x16, w1, w2)
    return _combine(pos, wts, ys)


# speedup vs baseline: 16.2188x; 1.5493x over previous
import functools

import jax
import jax.numpy as jnp
from jax.experimental import pallas as pl
from jax.experimental.pallas import tpu as pltpu

E = 8
TOPK = 2
D = 768
F = 768
T = 2048

_NEG = -1e30


def _moe_dense_body(x_ref, wg_ref, w1_ref, w2_ref, out_ref,
                    i1_ref, i2_ref, wa_ref, wb_ref, acc_ref, xb_ref):
    e = pl.program_id(0)

    @pl.when(e == 0)
    def _router():
        x = x_ref[...]
        xb_ref[...] = x.astype(jnp.bfloat16)
        logits = jnp.dot(x, wg_ref[...], preferred_element_type=jnp.float32)
        idx = jax.lax.broadcasted_iota(jnp.int32, (T, E), 1)
        m1 = jnp.max(logits, axis=1, keepdims=True)
        i1 = jnp.min(jnp.where(logits == m1, idx, E), axis=1, keepdims=True)
        l2 = jnp.where(idx == i1, _NEG, logits)
        m2 = jnp.max(l2, axis=1, keepdims=True)
        i2 = jnp.min(jnp.where(l2 == m2, idx, E), axis=1, keepdims=True)
        wa = jax.nn.sigmoid(m1 - m2)
        i1_ref[...] = i1
        i2_ref[...] = i2
        wa_ref[...] = wa
        wb_ref[...] = 1.0 - wa
        acc_ref[...] = jnp.zeros_like(acc_ref)

    coeff = (wa_ref[...] * (i1_ref[...] == e).astype(jnp.float32)
             + wb_ref[...] * (i2_ref[...] == e).astype(jnp.float32))

    w1e = w1_ref[0].astype(jnp.bfloat16)
    w2e = w2_ref[0].astype(jnp.bfloat16)
    HALF = T // 4
    for h in range(4):
        xb = xb_ref[h * HALF:(h + 1) * HALF, :]
        hh = jax.lax.dot_general(
            xb, w1e, (((1,), (1,)), ((), ())),
            preferred_element_type=jnp.float32)
        g = hh[:, :F]
        u = hh[:, F:]
        act = (g * jax.nn.sigmoid(g) * u).astype(jnp.bfloat16)
        y = jnp.dot(act, w2e, preferred_element_type=jnp.float32)
        acc_ref[h * HALF:(h + 1) * HALF, :] += coeff[h * HALF:(h + 1) * HALF, :] * y

    @pl.when(e == E - 1)
    def _emit():
        out_ref[...] = acc_ref[...]


@jax.jit
def kernel(hidden_states, Wg, w1, w2):
    return pl.pallas_call(
        _moe_dense_body,
        grid=(E,),
        in_specs=[
            pl.BlockSpec((T, D), lambda e: (0, 0)),
            pl.BlockSpec((D, E), lambda e: (0, 0)),
            pl.BlockSpec((1, 2 * F, D), lambda e: (e, 0, 0)),
            pl.BlockSpec((1, F, D), lambda e: (e, 0, 0)),
        ],
        out_specs=pl.BlockSpec((T, D), lambda e: (0, 0)),
        out_shape=jax.ShapeDtypeStruct((T, D), jnp.float32),
        scratch_shapes=[
            pltpu.VMEM((T, 1), jnp.int32),
            pltpu.VMEM((T, 1), jnp.int32),
            pltpu.VMEM((T, 1), jnp.float32),
            pltpu.VMEM((T, 1), jnp.float32),
            pltpu.VMEM((T, D), jnp.float32),
            pltpu.VMEM((T, D), jnp.bfloat16),
        ],
    )(hidden_states, Wg, w1, w2)


# speedup vs baseline: 16.3821x; 1.0101x over previous
import functools

import jax
import jax.numpy as jnp
from jax.experimental import pallas as pl
from jax.experimental.pallas import tpu as pltpu

E = 8
TOPK = 2
D = 768
F = 768
T = 2048

_NEG = -1e30


def _moe_dense_body(x_ref, wg_ref, w1_ref, w2_ref, out_ref,
                    i1_ref, i2_ref, wa_ref, wb_ref, acc_ref):
    e = pl.program_id(0)

    @pl.when(e == 0)
    def _router():
        x = x_ref[...]
        logits = jnp.dot(x, wg_ref[...], preferred_element_type=jnp.float32)
        idx = jax.lax.broadcasted_iota(jnp.int32, (T, E), 1)
        m1 = jnp.max(logits, axis=1, keepdims=True)
        i1 = jnp.min(jnp.where(logits == m1, idx, E), axis=1, keepdims=True)
        l2 = jnp.where(idx == i1, _NEG, logits)
        m2 = jnp.max(l2, axis=1, keepdims=True)
        i2 = jnp.min(jnp.where(l2 == m2, idx, E), axis=1, keepdims=True)
        wa = jax.nn.sigmoid(m1 - m2)
        i1_ref[...] = i1
        i2_ref[...] = i2
        wa_ref[...] = wa
        wb_ref[...] = 1.0 - wa
        acc_ref[...] = jnp.zeros_like(acc_ref)

    coeff = (wa_ref[...] * (i1_ref[...] == e).astype(jnp.float32)
             + wb_ref[...] * (i2_ref[...] == e).astype(jnp.float32))

    w1e = w1_ref[0].astype(jnp.bfloat16)
    w2e = w2_ref[0].astype(jnp.bfloat16)
    HALF = T // 2
    for h in range(2):
        xb = x_ref[h * HALF:(h + 1) * HALF, :].astype(jnp.bfloat16)
        hh = jax.lax.dot_general(
            xb, w1e, (((1,), (1,)), ((), ())),
            preferred_element_type=jnp.float32)
        g = hh[:, :F]
        u = hh[:, F:]
        act = (g * jax.nn.sigmoid(g) * u).astype(jnp.bfloat16)
        y = jnp.dot(act, w2e, preferred_element_type=jnp.float32)
        acc_ref[h * HALF:(h + 1) * HALF, :] += coeff[h * HALF:(h + 1) * HALF, :] * y

    @pl.when(e == E - 1)
    def _emit():
        out_ref[...] = acc_ref[...]


@jax.jit
def kernel(hidden_states, Wg, w1, w2):
    return pl.pallas_call(
        _moe_dense_body,
        grid=(E,),
        in_specs=[
            pl.BlockSpec((T, D), lambda e: (0, 0)),
            pl.BlockSpec((D, E), lambda e: (0, 0)),
            pl.BlockSpec((1, 2 * F, D), lambda e: (e, 0, 0)),
            pl.BlockSpec((1, F, D), lambda e: (e, 0, 0)),
        ],
        out_specs=pl.BlockSpec((T, D), lambda e: (0, 0)),
        out_shape=jax.ShapeDtypeStruct((T, D), jnp.float32),
        scratch_shapes=[
            pltpu.VMEM((T, 1), jnp.int32),
            pltpu.VMEM((T, 1), jnp.int32),
            pltpu.VMEM((T, 1), jnp.float32),
            pltpu.VMEM((T, 1), jnp.float32),
            pltpu.VMEM((T, D), jnp.float32),
        ],
    )(hidden_states, Wg, w1, w2)


# speedup vs baseline: 16.4438x; 1.0038x over previous
import functools

import jax
import jax.numpy as jnp
from jax.experimental import pallas as pl
from jax.experimental.pallas import tpu as pltpu

E = 8
TOPK = 2
D = 768
F = 768
T = 2048

_NEG = -1e30


def _moe_dense_body(x_ref, wg_ref, w1_ref, w2_ref, out_ref,
                    i1_ref, i2_ref, wa_ref, wb_ref):
    e = pl.program_id(0)

    @pl.when(e == 0)
    def _router():
        x = x_ref[...]
        logits = jnp.dot(x, wg_ref[...], preferred_element_type=jnp.float32)
        idx = jax.lax.broadcasted_iota(jnp.int32, (T, E), 1)
        m1 = jnp.max(logits, axis=1, keepdims=True)
        i1 = jnp.min(jnp.where(logits == m1, idx, E), axis=1, keepdims=True)
        l2 = jnp.where(idx == i1, _NEG, logits)
        m2 = jnp.max(l2, axis=1, keepdims=True)
        i2 = jnp.min(jnp.where(l2 == m2, idx, E), axis=1, keepdims=True)
        wa = jax.nn.sigmoid(m1 - m2)
        i1_ref[...] = i1
        i2_ref[...] = i2
        wa_ref[...] = wa
        wb_ref[...] = 1.0 - wa
        out_ref[...] = jnp.zeros_like(out_ref)

    coeff = (wa_ref[...] * (i1_ref[...] == e).astype(jnp.float32)
             + wb_ref[...] * (i2_ref[...] == e).astype(jnp.float32))

    w1e = w1_ref[0].astype(jnp.bfloat16)
    w2e = w2_ref[0].astype(jnp.bfloat16)
    HALF = T // 2
    for h in range(2):
        xb = x_ref[h * HALF:(h + 1) * HALF, :].astype(jnp.bfloat16)
        hh = jax.lax.dot_general(
            xb, w1e, (((1,), (1,)), ((), ())),
            preferred_element_type=jnp.float32)
        g = hh[:, :F]
        u = hh[:, F:]
        act = (g * jax.nn.sigmoid(g) * u).astype(jnp.bfloat16)
        y = jnp.dot(act, w2e, preferred_element_type=jnp.float32)
        out_ref[h * HALF:(h + 1) * HALF, :] += coeff[h * HALF:(h + 1) * HALF, :] * y


@jax.jit
def kernel(hidden_states, Wg, w1, w2):
    return pl.pallas_call(
        _moe_dense_body,
        grid=(E,),
        in_specs=[
            pl.BlockSpec((T, D), lambda e: (0, 0)),
            pl.BlockSpec((D, E), lambda e: (0, 0)),
            pl.BlockSpec((1, 2 * F, D), lambda e: (e, 0, 0)),
            pl.BlockSpec((1, F, D), lambda e: (e, 0, 0)),
        ],
        out_specs=pl.BlockSpec((T, D), lambda e: (0, 0)),
        out_shape=jax.ShapeDtypeStruct((T, D), jnp.float32),
        scratch_shapes=[
            pltpu.VMEM((T, 1), jnp.int32),
            pltpu.VMEM((T, 1), jnp.int32),
            pltpu.VMEM((T, 1), jnp.float32),
            pltpu.VMEM((T, 1), jnp.float32),
        ],
    )(hidden_states, Wg, w1, w2)
